# Initial kernel scaffold; baseline (speedup 1.0000x reference)
import jax
import jax.numpy as jnp
from jax.experimental import pallas as pl


def kernel(x, edge_index, W1, b1, W2, b2, W3, b3):
    raise NotImplementedError("write your pallas kernel here")


# speedup vs baseline: 46.3622x; 46.3622x over previous
import functools

import jax
import jax.numpy as jnp
from jax import lax
from jax.experimental import pallas as pl
from jax.experimental.pallas import tpu as pltpu
from jax.experimental.pallas import tpu_sc as plsc

NC = 2
NS = 16
NW = NC * NS
LN = 16
ROW = 128


def _mesh():
  return plsc.VectorSubcoreMesh(
      core_axis_name="c", subcore_axis_name="s",
      num_cores=NC, num_subcores=NS)


def _cdiv(a, b):
  return (a + b - 1) // b


def _fill(ref, n, value):
  v = jnp.full((LN,), value, ref.dtype)

  def body(i, _):
    ref[pl.ds(i * LN, LN)] = v
    return 0

  lax.fori_loop(0, n // LN, body, 0)


@functools.cache
def _degree_kernel(R, KB, NPAD):
  nchunk = R // KB
  rounds = _cdiv(nchunk, NW)
  sl = NPAD // NS

  def body(dst2d, out, acc, idx, ones, zbuf):
    c = lax.axis_index("c")
    s = lax.axis_index("s")
    w32 = c * NS + s
    _fill(ones, ROW, 1.0)
    _fill(zbuf, sl, 0.0)
    pltpu.sync_copy(zbuf, acc.at[pl.ds(s * sl, sl)])
    plsc.subcore_barrier()

    def round_body(k, _):
      cid = w32 + k * NW

      @pl.when(cid < nchunk)
      def _():
        pltpu.sync_copy(dst2d.at[pl.ds(cid * KB, KB)], idx)

        def row(j, _):
          pltpu.sync_copy(ones, acc.at[idx.at[j]], add=True)
          return 0

        lax.fori_loop(0, KB, row, 0)
      return 0

    lax.fori_loop(0, rounds, round_body, 0)
    plsc.subcore_barrier()
    pltpu.sync_copy(acc.at[pl.ds(s * sl, sl)], zbuf)
    pltpu.sync_copy(zbuf, out.at[pl.ds(c * NPAD + s * sl, sl)])

  return pl.kernel(
      body,
      out_type=jax.ShapeDtypeStruct((NC * NPAD,), jnp.float32),
      mesh=_mesh(),
      scratch_types=[
          pltpu.VMEM_SHARED((NPAD,), jnp.float32),
          pltpu.VMEM((KB, ROW), jnp.int32),
          pltpu.VMEM((ROW,), jnp.float32),
          pltpu.VMEM((sl,), jnp.float32),
      ],
  )


@functools.cache
def _conv_kernel(w, R, KB, NPAD):
  nchunk = R // KB
  rounds = _cdiv(nchunk, NW)
  sl = NPAD // NS

  def body(src2d, dst2d, *rest):
    us = rest[:w]
    outs = rest[w:2 * w]
    utab = rest[2 * w:3 * w]
    acc = rest[3 * w:4 * w]
    idxs, idxd, val, zbuf = rest[4 * w:4 * w + 4]
    c = lax.axis_index("c")
    s = lax.axis_index("s")
    w32 = c * NS + s
    _fill(zbuf, sl, 0.0)
    tsl = pl.ds(s * sl, sl)
    for cc in range(w):
      pltpu.sync_copy(zbuf, acc[cc].at[tsl])
    for cc in range(w):
      pltpu.sync_copy(us[cc].at[tsl], zbuf)
      pltpu.sync_copy(zbuf, utab[cc].at[tsl])
    plsc.subcore_barrier()

    def round_body(k, _):
      cid = w32 + k * NW

      @pl.when(cid < nchunk)
      def _():
        csl = pl.ds(cid * KB, KB)
        pltpu.sync_copy(src2d.at[csl], idxs)
        pltpu.sync_copy(dst2d.at[csl], idxd)

        def row(j, _):
          for cc in range(w):
            pltpu.sync_copy(utab[cc].at[idxs.at[j]], val)
            pltpu.sync_copy(val, acc[cc].at[idxd.at[j]], add=True)
          return 0

        lax.fori_loop(0, KB, row, 0)
      return 0

    lax.fori_loop(0, rounds, round_body, 0)
    plsc.subcore_barrier()
    osl = pl.ds(c * NPAD + s * sl, sl)
    for cc in range(w):
      pltpu.sync_copy(acc[cc].at[tsl], zbuf)
      pltpu.sync_copy(zbuf, outs[cc].at[osl])

  return pl.kernel(
      body,
      out_type=[jax.ShapeDtypeStruct((NC * NPAD,), jnp.float32)] * w,
      mesh=_mesh(),
      scratch_types=(
          [pltpu.VMEM_SHARED((NPAD,), jnp.float32)] * (2 * w) + [
              pltpu.VMEM((KB, ROW), jnp.int32),
              pltpu.VMEM((KB, ROW), jnp.int32),
              pltpu.VMEM((ROW,), jnp.float32),
              pltpu.VMEM((sl,), jnp.float32),
          ]),
  )


@functools.cache
def _readout_kernel(RH, KB, NPAD, EH):
  nchunk = RH // KB
  rounds = _cdiv(nchunk, NW)
  cb = KB * ROW
  sl = NPAD // NS

  def body(sa2d, da2d, sb2d, db2d, h0, h1, h2, h3, out,
           t0, t1, t2, t3, isa, ida, isb, idb, gbuf, prob, bounce):
    htab = (t0, t1, t2, t3)
    hs = (h0, h1, h2, h3)
    c = lax.axis_index("c")
    s = lax.axis_index("s")
    w32 = c * NS + s
    tsl = pl.ds(s * sl, sl)
    for cc in range(4):
      pltpu.sync_copy(hs[cc].at[tsl], bounce)
      pltpu.sync_copy(bounce, htab[cc].at[tsl])
    plsc.subcore_barrier()

    def round_body(k, _):
      cid = w32 + k * NW

      @pl.when(cid < nchunk)
      def _():
        csl = pl.ds(cid * KB, KB)
        pltpu.sync_copy(sa2d.at[csl], isa)
        pltpu.sync_copy(da2d.at[csl], ida)
        pltpu.sync_copy(sb2d.at[csl], isb)
        pltpu.sync_copy(db2d.at[csl], idb)

        def row(j, _):
          for cc in range(4):
            pltpu.sync_copy(htab[cc].at[isa.at[j]],
                            gbuf.at[pl.ds((4 * cc + 0) * ROW, ROW)])
            pltpu.sync_copy(htab[cc].at[ida.at[j]],
                            gbuf.at[pl.ds((4 * cc + 1) * ROW, ROW)])
            pltpu.sync_copy(htab[cc].at[isb.at[j]],
                            gbuf.at[pl.ds((4 * cc + 2) * ROW, ROW)])
            pltpu.sync_copy(htab[cc].at[idb.at[j]],
                            gbuf.at[pl.ds((4 * cc + 3) * ROW, ROW)])

          def vec(i, _):
            o = i * LN
            acc = jnp.zeros((LN,), jnp.float32)
            for cc in range(4):
              acc = acc + (gbuf[pl.ds((4 * cc + 0) * ROW + o, LN)] *
                           gbuf[pl.ds((4 * cc + 1) * ROW + o, LN)])
              acc = acc + (gbuf[pl.ds((4 * cc + 2) * ROW + o, LN)] *
                           gbuf[pl.ds((4 * cc + 3) * ROW + o, LN)])
            sv = acc * 0.5
            pv = 1.0 / (1.0 + jnp.exp(-sv))
            prob[pl.ds(j * ROW + o, LN)] = pv
            return 0

          lax.fori_loop(0, ROW // LN, vec, 0)
          return 0

        lax.fori_loop(0, KB, row, 0)
        pltpu.sync_copy(prob, out.at[pl.ds(cid * cb, cb)])
      return 0

    lax.fori_loop(0, rounds, round_body, 0)

  return pl.kernel(
      body,
      out_type=jax.ShapeDtypeStruct((EH,), jnp.float32),
      mesh=_mesh(),
      scratch_types=(
          [pltpu.VMEM_SHARED((NPAD,), jnp.float32)] * 4 + [
              pltpu.VMEM((KB, ROW), jnp.int32),
              pltpu.VMEM((KB, ROW), jnp.int32),
              pltpu.VMEM((KB, ROW), jnp.int32),
              pltpu.VMEM((KB, ROW), jnp.int32),
              pltpu.VMEM((16 * ROW,), jnp.float32),
              pltpu.VMEM((KB * ROW,), jnp.float32),
              pltpu.VMEM((sl,), jnp.float32),
          ]),
  )


def _pad(col, npad):
  n = col.shape[0]
  return jnp.concatenate([col, jnp.zeros((npad - n,), col.dtype)])


def kernel(x, edge_index, W1, b1, W2, b2, W3, b3):
  n = x.shape[0]
  e = edge_index.shape[1]
  eh = e // 2
  npad = _cdiv(n, NS * LN) * NS * LN
  r = e // ROW
  rh = eh // ROW

  src0 = edge_index[0]
  dst0 = edge_index[1]
  src2d = src0.reshape(r, ROW)
  dst2d = dst0.reshape(r, ROW)
  sa2d = src0[:eh].reshape(rh, ROW)
  da2d = dst0[:eh].reshape(rh, ROW)
  sb2d = src0[eh:].reshape(rh, ROW)
  db2d = dst0[eh:].reshape(rh, ROW)

  degp = _degree_kernel(r, 16, npad)(dst2d).reshape(NC, npad)
  deg = degp[0, :n] + degp[1, :n] + 1.0
  dinv = lax.rsqrt(deg)

  hw1 = x[:, 0] * W1[0, 0] + x[:, 1] * W1[1, 0] \
      + x[:, 2] * W1[2, 0] + x[:, 3] * W1[3, 0]
  u1 = dinv * hw1
  (p1,) = _conv_kernel(1, r, 16, npad)(src2d, dst2d, _pad(u1, npad))
  p1 = p1.reshape(NC, npad)
  h1 = jax.nn.relu(dinv * (p1[0, :n] + p1[1, :n] + u1) + b1[0])

  u2 = [dinv * (h1 * W2[0, cc]) for cc in range(2)]
  p2 = _conv_kernel(2, r, 16, npad)(
      src2d, dst2d, *[_pad(u, npad) for u in u2])
  p2 = [p.reshape(NC, npad) for p in p2]
  h2 = [jax.nn.relu(dinv * (p2[cc][0, :n] + p2[cc][1, :n] + u2[cc]) + b2[cc])
        for cc in range(2)]

  u3 = [dinv * (h2[0] * W3[0, cc] + h2[1] * W3[1, cc]) for cc in range(4)]
  p3 = _conv_kernel(4, r, 16, npad)(
      src2d, dst2d, *[_pad(u, npad) for u in u3])
  p3 = [p.reshape(NC, npad) for p in p3]
  h3 = [dinv * (p3[cc][0, :n] + p3[cc][1, :n] + u3[cc]) + b3[cc]
        for cc in range(4)]

  probs = _readout_kernel(rh, 8, npad, eh)(
      sa2d, da2d, sb2d, db2d, *[_pad(h, npad) for h in h3])
  return probs[:, None]


# speedup vs baseline: 111.8720x; 2.4130x over previous
import functools

import jax
import jax.numpy as jnp
from jax import lax
from jax.experimental import pallas as pl
from jax.experimental.pallas import tpu as pltpu
from jax.experimental.pallas import tpu_sc as plsc

NC = 2
NS = 16
NW = NC * NS
LN = 16
ROW = 128


def _mesh():
  return plsc.VectorSubcoreMesh(
      core_axis_name="c", subcore_axis_name="s",
      num_cores=NC, num_subcores=NS)


def _cdiv(a, b):
  return (a + b - 1) // b


def _fill(ref, n, value):
  v = jnp.full((LN,), value, ref.dtype)

  def body(i, _):
    ref[pl.ds(i * LN, LN)] = v
    return 0

  lax.fori_loop(0, n // LN, body, 0)


def _fill2d(ref, rows, cols, value):
  v = jnp.full((LN,), value, ref.dtype)

  def body(i, _):
    j = i // (cols // LN)
    o = (i % (cols // LN)) * LN
    ref[j, pl.ds(o, LN)] = v
    return 0

  lax.fori_loop(0, rows * (cols // LN), body, 0)


@functools.cache
def _degree_kernel(R, KB, NPAD):
  nchunk = R // KB
  rounds = _cdiv(nchunk, NW)
  sl = NPAD // NS

  def body(dst2d, out, acc, idx, ones, zbuf, sems):
    c = lax.axis_index("c")
    s = lax.axis_index("s")
    w32 = c * NS + s
    _fill(ones, ROW, 1.0)
    _fill(zbuf, sl, 0.0)
    pltpu.sync_copy(zbuf, acc.at[pl.ds(s * sl, sl)])
    plsc.subcore_barrier()

    def round_body(k, _):
      cid = w32 + k * NW

      @pl.when(cid < nchunk)
      def _():
        pltpu.sync_copy(dst2d.at[pl.ds(cid * KB, KB)], idx)
        descs = [pltpu.async_copy(ones, acc.at[idx.at[j]], sems, add=True)
                 for j in range(KB)]
        for d in descs:
          d.wait()
      return 0

    lax.fori_loop(0, rounds, round_body, 0)
    plsc.subcore_barrier()
    pltpu.sync_copy(acc.at[pl.ds(s * sl, sl)], zbuf)
    pltpu.sync_copy(zbuf, out.at[pl.ds(c * NPAD + s * sl, sl)])

  return pl.kernel(
      body,
      out_type=jax.ShapeDtypeStruct((NC * NPAD,), jnp.float32),
      mesh=_mesh(),
      scratch_types=[
          pltpu.VMEM_SHARED((NPAD,), jnp.float32),
          pltpu.VMEM((KB, ROW), jnp.int32),
          pltpu.VMEM((ROW,), jnp.float32),
          pltpu.VMEM((sl,), jnp.float32),
          pltpu.SemaphoreType.DMA,
      ],
  )


@functools.cache
def _conv_kernel(w, R, KB, NPAD):
  nchunk = R // KB
  rounds = _cdiv(nchunk, NW)
  sl = NPAD // NS

  def body(src2d, dst2d, *rest):
    us = rest[:w]
    outs = rest[w:2 * w]
    utab = rest[2 * w:3 * w]
    acc = rest[3 * w:4 * w]
    idxs, idxd, val, zbuf, semg, sems = rest[4 * w:4 * w + 6]
    c = lax.axis_index("c")
    s = lax.axis_index("s")
    w32 = c * NS + s
    _fill(zbuf, sl, 0.0)
    tsl = pl.ds(s * sl, sl)
    for cc in range(w):
      pltpu.sync_copy(zbuf, acc[cc].at[tsl])
    for cc in range(w):
      pltpu.sync_copy(us[cc].at[tsl], zbuf)
      pltpu.sync_copy(zbuf, utab[cc].at[tsl])
    plsc.subcore_barrier()

    def round_body(k, _):
      cid = w32 + k * NW

      @pl.when(cid < nchunk)
      def _():
        csl = pl.ds(cid * KB, KB)
        pltpu.sync_copy(src2d.at[csl], idxs)
        pltpu.sync_copy(dst2d.at[csl], idxd)
        descs = [
            pltpu.async_copy(utab[cc].at[idxs.at[j]], val.at[cc * KB + j],
                             semg)
            for j in range(KB) for cc in range(w)]
        for d in descs:
          d.wait()
        descs = [
            pltpu.async_copy(val.at[cc * KB + j], acc[cc].at[idxd.at[j]],
                             sems, add=True)
            for j in range(KB) for cc in range(w)]
        for d in descs:
          d.wait()
      return 0

    lax.fori_loop(0, rounds, round_body, 0)
    plsc.subcore_barrier()
    osl = pl.ds(c * NPAD + s * sl, sl)
    for cc in range(w):
      pltpu.sync_copy(acc[cc].at[tsl], zbuf)
      pltpu.sync_copy(zbuf, outs[cc].at[osl])

  return pl.kernel(
      body,
      out_type=[jax.ShapeDtypeStruct((NC * NPAD,), jnp.float32)] * w,
      mesh=_mesh(),
      scratch_types=(
          [pltpu.VMEM_SHARED((NPAD,), jnp.float32)] * (2 * w) + [
              pltpu.VMEM((KB, ROW), jnp.int32),
              pltpu.VMEM((KB, ROW), jnp.int32),
              pltpu.VMEM((w * KB, ROW), jnp.float32),
              pltpu.VMEM((sl,), jnp.float32),
              pltpu.SemaphoreType.DMA,
              pltpu.SemaphoreType.DMA,
          ]),
  )


@functools.cache
def _readout_kernel(RH, KB, NPAD, EH):
  nchunk = RH // KB
  rounds = _cdiv(nchunk, NW)
  cb = KB * ROW
  sl = NPAD // NS

  def body(sa2d, da2d, sb2d, db2d, h0, h1, h2, h3, out,
           t0, t1, t2, t3, isa, ida, isb, idb, gbuf, prob, bounce, semg):
    htab = (t0, t1, t2, t3)
    hs = (h0, h1, h2, h3)
    c = lax.axis_index("c")
    s = lax.axis_index("s")
    w32 = c * NS + s
    tsl = pl.ds(s * sl, sl)
    for cc in range(4):
      pltpu.sync_copy(hs[cc].at[tsl], bounce)
      pltpu.sync_copy(bounce, htab[cc].at[tsl])
    plsc.subcore_barrier()

    def round_body(k, _):
      cid = w32 + k * NW

      @pl.when(cid < nchunk)
      def _():
        csl = pl.ds(cid * KB, KB)
        pltpu.sync_copy(sa2d.at[csl], isa)
        pltpu.sync_copy(da2d.at[csl], ida)
        pltpu.sync_copy(sb2d.at[csl], isb)
        pltpu.sync_copy(db2d.at[csl], idb)
        idrefs = (isa, ida, isb, idb)
        descs = [
            pltpu.async_copy(htab[cc].at[idrefs[t].at[j]],
                             gbuf.at[(4 * cc + t) * KB + j], semg)
            for j in range(KB) for cc in range(4) for t in range(4)]
        for d in descs:
          d.wait()

        def row(j, _):
          for i in range(ROW // LN):
            o = i * LN
            acc = jnp.zeros((LN,), jnp.float32)
            for cc in range(4):
              acc = acc + (gbuf[(4 * cc + 0) * KB + j, pl.ds(o, LN)] *
                           gbuf[(4 * cc + 1) * KB + j, pl.ds(o, LN)])
              acc = acc + (gbuf[(4 * cc + 2) * KB + j, pl.ds(o, LN)] *
                           gbuf[(4 * cc + 3) * KB + j, pl.ds(o, LN)])
            sv = acc * 0.5
            pv = 1.0 / (1.0 + jnp.exp(-sv))
            prob[pl.ds(j * ROW + o, LN)] = pv
          return 0

        lax.fori_loop(0, KB, row, 0)
        pltpu.sync_copy(prob, out.at[pl.ds(cid * cb, cb)])
      return 0

    lax.fori_loop(0, rounds, round_body, 0)

  return pl.kernel(
      body,
      out_type=jax.ShapeDtypeStruct((EH,), jnp.float32),
      mesh=_mesh(),
      scratch_types=(
          [pltpu.VMEM_SHARED((NPAD,), jnp.float32)] * 4 + [
              pltpu.VMEM((KB, ROW), jnp.int32),
              pltpu.VMEM((KB, ROW), jnp.int32),
              pltpu.VMEM((KB, ROW), jnp.int32),
              pltpu.VMEM((KB, ROW), jnp.int32),
              pltpu.VMEM((16 * KB, ROW), jnp.float32),
              pltpu.VMEM((KB * ROW,), jnp.float32),
              pltpu.VMEM((sl,), jnp.float32),
              pltpu.SemaphoreType.DMA,
          ]),
  )


def _pad(col, npad):
  n = col.shape[0]
  return jnp.concatenate([col, jnp.zeros((npad - n,), col.dtype)])


def kernel(x, edge_index, W1, b1, W2, b2, W3, b3):
  n = x.shape[0]
  e = edge_index.shape[1]
  eh = e // 2
  npad = _cdiv(n, NS * LN) * NS * LN
  r = e // ROW
  rh = eh // ROW

  src0 = edge_index[0]
  dst0 = edge_index[1]
  src2d = src0.reshape(r, ROW)
  dst2d = dst0.reshape(r, ROW)
  sa2d = src0[:eh].reshape(rh, ROW)
  da2d = dst0[:eh].reshape(rh, ROW)
  sb2d = src0[eh:].reshape(rh, ROW)
  db2d = dst0[eh:].reshape(rh, ROW)

  degp = _degree_kernel(r, 16, npad)(dst2d).reshape(NC, npad)
  deg = degp[0, :n] + degp[1, :n] + 1.0
  dinv = lax.rsqrt(deg)

  hw1 = x[:, 0] * W1[0, 0] + x[:, 1] * W1[1, 0] \
      + x[:, 2] * W1[2, 0] + x[:, 3] * W1[3, 0]
  u1 = dinv * hw1
  (p1,) = _conv_kernel(1, r, 16, npad)(src2d, dst2d, _pad(u1, npad))
  p1 = p1.reshape(NC, npad)
  h1 = jax.nn.relu(dinv * (p1[0, :n] + p1[1, :n] + u1) + b1[0])

  u2 = [dinv * (h1 * W2[0, cc]) for cc in range(2)]
  p2 = _conv_kernel(2, r, 16, npad)(
      src2d, dst2d, *[_pad(u, npad) for u in u2])
  p2 = [p.reshape(NC, npad) for p in p2]
  h2 = [jax.nn.relu(dinv * (p2[cc][0, :n] + p2[cc][1, :n] + u2[cc]) + b2[cc])
        for cc in range(2)]

  u3 = [dinv * (h2[0] * W3[0, cc] + h2[1] * W3[1, cc]) for cc in range(4)]
  p3 = _conv_kernel(4, r, 16, npad)(
      src2d, dst2d, *[_pad(u, npad) for u in u3])
  p3 = [p.reshape(NC, npad) for p in p3]
  h3 = [dinv * (p3[cc][0, :n] + p3[cc][1, :n] + u3[cc]) + b3[cc]
        for cc in range(4)]

  probs = _readout_kernel(rh, 8, npad, eh)(
      sa2d, da2d, sb2d, db2d, *[_pad(h, npad) for h in h3])
  return probs[:, None]
